# baseline (speedup 1.0000x reference)
import functools

import jax
import jax.numpy as jnp
from jax import lax
from jax.experimental import pallas as pl
from jax.experimental.pallas import tpu as pltpu
from jax.experimental.pallas import tpu_sc as plsc

_TOPK = 2
_CAP_FACTOR = 1.5

_NC = 2
_NS = 16
_L = 16


def _router_body(x_ref, w1_ref, b1_ref, w2_ref, b2_ref,
                 probs_ref, pres_ref, psum_ref):
    i = pl.program_id(0)
    x = x_ref[...]
    h = jnp.maximum(
        jnp.dot(x, w1_ref[...], preferred_element_type=jnp.float32)
        + b1_ref[...], 0.0)
    logits = (jnp.dot(h, w2_ref[...], preferred_element_type=jnp.float32)
              + b2_ref[...])
    m = jnp.max(logits, axis=-1, keepdims=True)
    ex = jnp.exp(logits - m)
    probs = ex / jnp.sum(ex, axis=-1, keepdims=True)
    probs_ref[...] = probs

    tn, n_e = probs.shape
    eio = jax.lax.broadcasted_iota(
        jnp.int32, (tn, n_e), 1).astype(jnp.float32)
    p0 = jnp.max(probs, axis=-1, keepdims=True)
    e0 = jnp.min(jnp.where(probs == p0, eio, float(n_e)),
                 axis=-1, keepdims=True)
    blk_pres = jnp.max((eio == e0).astype(jnp.float32),
                       axis=0, keepdims=True)
    blk_psum = jnp.sum(probs, axis=0, keepdims=True)
    pad = jnp.zeros((1, 16 - n_e), jnp.float32)
    blk_pres16 = jnp.concatenate([blk_pres, pad], axis=1)

    @pl.when(i == 0)
    def _():
        pres_ref[...] = blk_pres16
        psum_ref[...] = blk_psum

    @pl.when(i > 0)
    def _():
        pres_ref[...] = jnp.maximum(pres_ref[...], blk_pres16)
        psum_ref[...] = psum_ref[...] + blk_psum


def _make_sc_router(n, n_e):
    tok_per_w = n // (_NC * _NS)
    groups = tok_per_w // _L
    mesh = plsc.VectorSubcoreMesh(core_axis_name="c", subcore_axis_name="s")

    @functools.partial(
        pl.kernel, mesh=mesh,
        out_type=jax.ShapeDtypeStruct((n * 8,), jnp.float32),
        scratch_types=[
            pltpu.VMEM((tok_per_w * n_e,), jnp.float32),
            pltpu.VMEM((tok_per_w * 8,), jnp.float32),
            pltpu.VMEM((16,), jnp.float32),
        ],
    )
    def sc_route(probsG_hbm, pres_hbm, meta_hbm, pbuf, mchunk, presv):
        wid = lax.axis_index("s") * _NC + lax.axis_index("c")
        csz = tok_per_w * 8
        pltpu.sync_copy(probsG_hbm.at[pl.ds(wid * csz, csz)], pbuf)
        pltpu.sync_copy(pres_hbm, presv)
        presvec = presv[...]
        for g in range(groups):
            sl = pl.ds(g * _L, _L)
            pv = [pbuf[pl.ds(e * tok_per_w + g * _L, _L)] for e in range(n_e)]
            p0 = pv[0]
            e0 = jnp.zeros((_L,), jnp.float32)
            for e in range(1, n_e):
                mgt = pv[e] > p0
                p0 = jnp.where(mgt, pv[e], p0)
                e0 = jnp.where(mgt, float(e), e0)
            p1 = jnp.full((_L,), -1.0, jnp.float32)
            e1 = jnp.zeros((_L,), jnp.float32)
            for e in range(n_e):
                mgt = (e0 != float(e)) & (pv[e] > p1)
                p1 = jnp.where(mgt, pv[e], p1)
                e1 = jnp.where(mgt, float(e), e1)
            s = p0 + p1
            pos1 = presvec.at[e1.astype(jnp.int32)].get(
                mode="promise_in_bounds")
            def fsl(f):
                return pl.ds(f * tok_per_w + g * _L, _L)
            mchunk[fsl(0)] = e0
            mchunk[fsl(1)] = e1
            mchunk[fsl(2)] = p0 / s
            mchunk[fsl(3)] = p1 / s
            mchunk[fsl(4)] = pos1
            zl = jnp.zeros((_L,), jnp.float32)
            mchunk[fsl(5)] = zl
            mchunk[fsl(6)] = zl
            mchunk[fsl(7)] = zl
        pltpu.sync_copy(mchunk, meta_hbm.at[pl.ds(wid * csz, csz)])

    return sc_route


def _scatter_body(meta_ref, psum_ref, disp_ref, comb_ref, aux_ref,
                  *, cap, n_tokens):
    i = pl.program_id(0)
    meta = meta_ref[...]
    tm = meta.shape[0]
    n_e = psum_ref.shape[1]
    e0 = meta[:, 0].reshape(tm, 1, 1)
    e1 = meta[:, 1].reshape(tm, 1, 1)
    p0 = meta[:, 2].reshape(tm, 1, 1)
    p1 = meta[:, 3].reshape(tm, 1, 1)
    pos1 = meta[:, 4].reshape(tm, 1, 1)

    w = 128
    e_ax = jax.lax.broadcasted_iota(
        jnp.int32, (tm, n_e, w), 1).astype(jnp.float32)
    c_ax = jax.lax.broadcasted_iota(
        jnp.int32, (tm, n_e, w), 2).astype(jnp.float32)
    hit0 = (e_ax == e0) & (c_ax == 0.0)
    hit1 = (e_ax == e1) & (c_ax == pos1)
    disp_ref[:, :, 0:w] = hit0.astype(jnp.float32) + hit1.astype(jnp.float32)
    comb_ref[:, :, 0:w] = (jnp.where(hit0, p0, 0.0)
                           + jnp.where(hit1, p1, 0.0))
    zt = jnp.zeros((tm, n_e, cap - w), jnp.float32)
    disp_ref[:, :, w:cap] = zt
    comb_ref[:, :, w:cap] = zt

    @pl.when(i == 0)
    def _():
        mean_p = psum_ref[...] * (1.0 / n_tokens)
        aux = jnp.sum(mean_p * jnp.log(mean_p * n_e + 1e-9))
        aux_ref[...] = aux.reshape(1, 1)


def kernel(hidden_states, W1, b1, W2, b2):
    b, s, h = hidden_states.shape
    e = W2.shape[1]
    n = b * s
    cap = int(b * s * _CAP_FACTOR * _TOPK / e)
    x = hidden_states.reshape(n, h)

    tb = 256
    probs, pres16, psum = pl.pallas_call(
        _router_body,
        grid=(n // tb,),
        in_specs=[
            pl.BlockSpec((tb, h), lambda i: (i, 0)),
            pl.BlockSpec((h, h), lambda i: (0, 0)),
            pl.BlockSpec((1, h), lambda i: (0, 0)),
            pl.BlockSpec((h, e), lambda i: (0, 0)),
            pl.BlockSpec((1, e), lambda i: (0, 0)),
        ],
        out_specs=[
            pl.BlockSpec((tb, e), lambda i: (i, 0)),
            pl.BlockSpec((1, 16), lambda i: (0, 0)),
            pl.BlockSpec((1, e), lambda i: (0, 0)),
        ],
        out_shape=[
            jax.ShapeDtypeStruct((n, e), jnp.float32),
            jax.ShapeDtypeStruct((1, 16), jnp.float32),
            jax.ShapeDtypeStruct((1, e), jnp.float32),
        ],
    )(x, W1, b1.reshape(1, h), W2, b2.reshape(1, e))

    nw = _NC * _NS
    tpw = n // nw
    probsG = probs.reshape(nw, tpw, e).transpose(0, 2, 1).reshape(n * e)
    meta = _make_sc_router(n, e)(probsG, pres16.reshape(16))
    meta = meta.reshape(nw, 8, tpw).transpose(0, 2, 1).reshape(n, 8)

    tm = 256
    disp, comb, aux = pl.pallas_call(
        functools.partial(_scatter_body, cap=cap, n_tokens=n),
        grid=(n // tm,),
        in_specs=[
            pl.BlockSpec((tm, 8), lambda i: (i, 0)),
            pl.BlockSpec((1, e), lambda i: (0, 0)),
        ],
        out_specs=[
            pl.BlockSpec((tm, e, cap), lambda i: (i, 0, 0)),
            pl.BlockSpec((tm, e, cap), lambda i: (i, 0, 0)),
            pl.BlockSpec((1, 1), lambda i: (0, 0)),
        ],
        out_shape=[
            jax.ShapeDtypeStruct((n, e, cap), jnp.float32),
            jax.ShapeDtypeStruct((n, e, cap), jnp.float32),
            jax.ShapeDtypeStruct((1, 1), jnp.float32),
        ],
    )(meta, psum)

    return (disp.reshape(b, s, e, cap), comb.reshape(b, s, e, cap),
            probs.reshape(b, s, e), aux[0, 0])


# speedup vs baseline: 1.0167x; 1.0167x over previous
import functools

import jax
import jax.numpy as jnp
from jax import lax
from jax.experimental import pallas as pl
from jax.experimental.pallas import tpu as pltpu
from jax.experimental.pallas import tpu_sc as plsc

_TOPK = 2
_CAP_FACTOR = 1.5

_NC = 2
_NS = 16
_L = 16


def _router_body(x_ref, w1_ref, b1_ref, w2_ref, b2_ref,
                 probs_ref, pres_ref, psum_ref):
    i = pl.program_id(0)
    x = x_ref[...]
    h = jnp.maximum(
        jnp.dot(x, w1_ref[...], preferred_element_type=jnp.float32)
        + b1_ref[...], 0.0)
    logits = (jnp.dot(h, w2_ref[...], preferred_element_type=jnp.float32)
              + b2_ref[...])
    m = jnp.max(logits, axis=-1, keepdims=True)
    ex = jnp.exp(logits - m)
    probs = ex / jnp.sum(ex, axis=-1, keepdims=True)
    probs_ref[...] = probs

    tn, n_e = probs.shape
    eio = jax.lax.broadcasted_iota(
        jnp.int32, (tn, n_e), 1).astype(jnp.float32)
    p0 = jnp.max(probs, axis=-1, keepdims=True)
    e0 = jnp.min(jnp.where(probs == p0, eio, float(n_e)),
                 axis=-1, keepdims=True)
    blk_pres = jnp.max((eio == e0).astype(jnp.float32),
                       axis=0, keepdims=True)
    blk_psum = jnp.sum(probs, axis=0, keepdims=True)
    pad = jnp.zeros((1, 16 - n_e), jnp.float32)
    blk_pres16 = jnp.concatenate([blk_pres, pad], axis=1)

    @pl.when(i == 0)
    def _():
        pres_ref[...] = blk_pres16
        psum_ref[...] = blk_psum

    @pl.when(i > 0)
    def _():
        pres_ref[...] = jnp.maximum(pres_ref[...], blk_pres16)
        psum_ref[...] = psum_ref[...] + blk_psum


def _make_sc_router(n, n_e):
    tok_per_w = n // (_NC * _NS)
    groups = tok_per_w // _L
    mesh = plsc.VectorSubcoreMesh(core_axis_name="c", subcore_axis_name="s")

    @functools.partial(
        pl.kernel, mesh=mesh,
        out_type=jax.ShapeDtypeStruct((n * 8,), jnp.float32),
        scratch_types=[
            pltpu.VMEM((tok_per_w * n_e,), jnp.float32),
            pltpu.VMEM((tok_per_w * 8,), jnp.float32),
            pltpu.VMEM((16,), jnp.float32),
        ],
    )
    def sc_route(probsG_hbm, pres_hbm, meta_hbm, pbuf, mchunk, presv):
        wid = lax.axis_index("s") * _NC + lax.axis_index("c")
        csz = tok_per_w * 8
        pltpu.sync_copy(probsG_hbm.at[pl.ds(wid * csz, csz)], pbuf)
        pltpu.sync_copy(pres_hbm, presv)
        presvec = presv[...]
        for g in range(groups):
            pv = [pbuf[pl.ds(e * tok_per_w + g * _L, _L)] for e in range(n_e)]
            p0 = pv[0]
            e0 = jnp.zeros((_L,), jnp.float32)
            for e in range(1, n_e):
                mgt = pv[e] > p0
                p0 = jnp.where(mgt, pv[e], p0)
                e0 = jnp.where(mgt, float(e), e0)
            p1 = jnp.full((_L,), -1.0, jnp.float32)
            e1 = jnp.zeros((_L,), jnp.float32)
            for e in range(n_e):
                mgt = (e0 != float(e)) & (pv[e] > p1)
                p1 = jnp.where(mgt, pv[e], p1)
                e1 = jnp.where(mgt, float(e), e1)
            s = p0 + p1
            pos1 = presvec.at[e1.astype(jnp.int32)].get(
                mode="promise_in_bounds")
            def fsl(f):
                return pl.ds(f * tok_per_w + g * _L, _L)
            mchunk[fsl(0)] = e0
            mchunk[fsl(1)] = e1
            mchunk[fsl(2)] = p0 / s
            mchunk[fsl(3)] = p1 / s
            mchunk[fsl(4)] = pos1
            zl = jnp.zeros((_L,), jnp.float32)
            mchunk[fsl(5)] = zl
            mchunk[fsl(6)] = zl
            mchunk[fsl(7)] = zl
        pltpu.sync_copy(mchunk, meta_hbm.at[pl.ds(wid * csz, csz)])

    return sc_route


def _scatter_body(meta_ref, psum_ref, disp_ref, comb_ref, aux_ref,
                  *, cap, n_tokens):
    i = pl.program_id(0)
    meta = meta_ref[...]
    tm = meta.shape[0]
    n_e = psum_ref.shape[1]
    e0 = meta[:, 0].reshape(tm, 1, 1)
    e1 = meta[:, 1].reshape(tm, 1, 1)
    p0 = meta[:, 2].reshape(tm, 1, 1)
    p1 = meta[:, 3].reshape(tm, 1, 1)
    pos1 = meta[:, 4].reshape(tm, 1, 1)

    w = 128
    e_ax = jax.lax.broadcasted_iota(
        jnp.int32, (tm, n_e, w), 1).astype(jnp.float32)
    c_ax = jax.lax.broadcasted_iota(
        jnp.int32, (tm, n_e, w), 2).astype(jnp.float32)
    hit0 = (e_ax == e0) & (c_ax == 0.0)
    hit1 = (e_ax == e1) & (c_ax == pos1)
    disp_ref[:, :, 0:w] = hit0.astype(jnp.float32) + hit1.astype(jnp.float32)
    comb_ref[:, :, 0:w] = (jnp.where(hit0, p0, 0.0)
                           + jnp.where(hit1, p1, 0.0))
    zt = jnp.zeros((tm, n_e, cap - w), jnp.float32)
    disp_ref[:, :, w:cap] = zt
    comb_ref[:, :, w:cap] = zt

    @pl.when(i == 0)
    def _():
        mean_p = psum_ref[...] * (1.0 / n_tokens)
        aux = jnp.sum(mean_p * jnp.log(mean_p * n_e + 1e-9))
        aux_ref[...] = aux.reshape(1, 1)


def kernel(hidden_states, W1, b1, W2, b2):
    b, s, h = hidden_states.shape
    e = W2.shape[1]
    n = b * s
    cap = int(b * s * _CAP_FACTOR * _TOPK / e)
    x = hidden_states.reshape(n, h)

    tb = 256
    probs, pres16, psum = pl.pallas_call(
        _router_body,
        grid=(n // tb,),
        in_specs=[
            pl.BlockSpec((tb, h), lambda i: (i, 0)),
            pl.BlockSpec((h, h), lambda i: (0, 0)),
            pl.BlockSpec((1, h), lambda i: (0, 0)),
            pl.BlockSpec((h, e), lambda i: (0, 0)),
            pl.BlockSpec((1, e), lambda i: (0, 0)),
        ],
        out_specs=[
            pl.BlockSpec((tb, e), lambda i: (i, 0)),
            pl.BlockSpec((1, 16), lambda i: (0, 0)),
            pl.BlockSpec((1, e), lambda i: (0, 0)),
        ],
        out_shape=[
            jax.ShapeDtypeStruct((n, e), jnp.float32),
            jax.ShapeDtypeStruct((1, 16), jnp.float32),
            jax.ShapeDtypeStruct((1, e), jnp.float32),
        ],
    )(x, W1, b1.reshape(1, h), W2, b2.reshape(1, e))

    nw = _NC * _NS
    tpw = n // nw
    probsG = probs.reshape(nw, tpw, e).transpose(0, 2, 1).reshape(n * e)
    meta = _make_sc_router(n, e)(probsG, pres16.reshape(16))
    meta = meta.reshape(nw, 8, tpw).transpose(0, 2, 1).reshape(n, 8)

    tm = 128
    disp, comb, aux = pl.pallas_call(
        functools.partial(_scatter_body, cap=cap, n_tokens=n),
        grid=(n // tm,),
        in_specs=[
            pl.BlockSpec((tm, 8), lambda i: (i, 0)),
            pl.BlockSpec((1, e), lambda i: (0, 0)),
        ],
        out_specs=[
            pl.BlockSpec((tm, e, cap), lambda i: (i, 0, 0)),
            pl.BlockSpec((tm, e, cap), lambda i: (i, 0, 0)),
            pl.BlockSpec((1, 1), lambda i: (0, 0)),
        ],
        out_shape=[
            jax.ShapeDtypeStruct((n, e, cap), jnp.float32),
            jax.ShapeDtypeStruct((n, e, cap), jnp.float32),
            jax.ShapeDtypeStruct((1, 1), jnp.float32),
        ],
    )(meta, psum)

    return (disp.reshape(b, s, e, cap), comb.reshape(b, s, e, cap),
            probs.reshape(b, s, e), aux[0, 0])


# speedup vs baseline: 1.0874x; 1.0695x over previous
import functools

import jax
import jax.numpy as jnp
from jax import lax
from jax.experimental import pallas as pl
from jax.experimental.pallas import tpu as pltpu
from jax.experimental.pallas import tpu_sc as plsc

_TOPK = 2
_CAP_FACTOR = 1.5

_NC = 2
_NS = 16
_L = 16


def _router_body(x_ref, w1_ref, b1_ref, w2_ref, b2_ref,
                 probs_ref, probsP_ref, pres_ref, psum_ref):
    i = pl.program_id(0)
    x = x_ref[...]
    h = jnp.maximum(
        jnp.dot(x, w1_ref[...], preferred_element_type=jnp.float32)
        + b1_ref[...], 0.0)
    logits = (jnp.dot(h, w2_ref[...], preferred_element_type=jnp.float32)
              + b2_ref[...])
    m = jnp.max(logits, axis=-1, keepdims=True)
    ex = jnp.exp(logits - m)
    probs = ex / jnp.sum(ex, axis=-1, keepdims=True)
    probs_ref[...] = probs

    tn, n_e = probs.shape
    eio = jax.lax.broadcasted_iota(
        jnp.int32, (tn, n_e), 1).astype(jnp.float32)
    p0 = jnp.max(probs, axis=-1, keepdims=True)
    e0 = jnp.min(jnp.where(probs == p0, eio, float(n_e)),
                 axis=-1, keepdims=True)
    blk_pres = jnp.max((eio == e0).astype(jnp.float32),
                       axis=0, keepdims=True)
    blk_psum = jnp.sum(probs, axis=0, keepdims=True)
    pad = jnp.zeros((1, 16 - n_e), jnp.float32)
    blk_pres16 = jnp.concatenate([blk_pres, pad], axis=1)
    probsP_ref[...] = jnp.transpose(probs.reshape(2, 128, n_e), (0, 2, 1))

    @pl.when(i == 0)
    def _():
        pres_ref[...] = blk_pres16
        psum_ref[...] = blk_psum

    @pl.when(i > 0)
    def _():
        pres_ref[...] = jnp.maximum(pres_ref[...], blk_pres16)
        psum_ref[...] = psum_ref[...] + blk_psum


def _make_sc_router(n, n_e):
    tok_per_p = 128
    npair = n // tok_per_p
    groups = tok_per_p // _L
    mesh = plsc.VectorSubcoreMesh(core_axis_name="c", subcore_axis_name="s")

    def _route_chunk(probsP_hbm, pres_hbm, meta_hbm, pbuf, mchunk,
                     presv, wid):
        pltpu.sync_copy(probsP_hbm.at[wid], pbuf)
        pltpu.sync_copy(pres_hbm, presv)
        presvec = presv[...]
        for g in range(groups):
            pv = [pbuf[e, pl.ds(g * _L, _L)] for e in range(n_e)]
            p0 = pv[0]
            e0 = jnp.zeros((_L,), jnp.float32)
            for e in range(1, n_e):
                mgt = pv[e] > p0
                p0 = jnp.where(mgt, pv[e], p0)
                e0 = jnp.where(mgt, float(e), e0)
            p1 = jnp.full((_L,), -1.0, jnp.float32)
            e1 = jnp.zeros((_L,), jnp.float32)
            for e in range(n_e):
                mgt = (e0 != float(e)) & (pv[e] > p1)
                p1 = jnp.where(mgt, pv[e], p1)
                e1 = jnp.where(mgt, float(e), e1)
            s = p0 + p1
            pos1 = presvec.at[e1.astype(jnp.int32)].get(
                mode="promise_in_bounds")
            sl = pl.ds(g * _L, _L)
            mchunk[0, sl] = e0
            mchunk[1, sl] = e1
            mchunk[2, sl] = p0 / s
            mchunk[3, sl] = p1 / s
            mchunk[4, sl] = pos1
            zl = jnp.zeros((_L,), jnp.float32)
            mchunk[5, sl] = zl
            mchunk[6, sl] = zl
            mchunk[7, sl] = zl
        pltpu.sync_copy(mchunk, meta_hbm.at[wid])

    @functools.partial(
        pl.kernel, mesh=mesh,
        out_type=jax.ShapeDtypeStruct((npair, 8, tok_per_p), jnp.float32),
        scratch_types=[
            pltpu.VMEM((n_e, tok_per_p), jnp.float32),
            pltpu.VMEM((8, tok_per_p), jnp.float32),
            pltpu.VMEM((16,), jnp.float32),
        ],
    )
    def sc_route(probsP_hbm, pres_hbm, meta_hbm, pbuf, mchunk, presv):
        wid = lax.axis_index("s") * _NC + lax.axis_index("c")

        @pl.when(wid < npair)
        def _():
            _route_chunk(probsP_hbm, pres_hbm, meta_hbm, pbuf, mchunk,
                         presv, wid)

    return sc_route


def _scatter_body(meta_ref, psum_ref, disp_ref, comb_ref, aux_ref,
                  *, cap, n_tokens):
    i = pl.program_id(0)
    meta = jnp.transpose(meta_ref[0, :, :])
    tm = meta.shape[0]
    n_e = psum_ref.shape[1]
    e0 = meta[:, 0].reshape(tm, 1, 1)
    e1 = meta[:, 1].reshape(tm, 1, 1)
    p0 = meta[:, 2].reshape(tm, 1, 1)
    p1 = meta[:, 3].reshape(tm, 1, 1)
    pos1 = meta[:, 4].reshape(tm, 1, 1)

    w = 128
    e_ax = jax.lax.broadcasted_iota(
        jnp.int32, (tm, n_e, w), 1).astype(jnp.float32)
    c_ax = jax.lax.broadcasted_iota(
        jnp.int32, (tm, n_e, w), 2).astype(jnp.float32)
    hit0 = (e_ax == e0) & (c_ax == 0.0)
    hit1 = (e_ax == e1) & (c_ax == pos1)
    disp_ref[:, :, 0:w] = hit0.astype(jnp.float32) + hit1.astype(jnp.float32)
    comb_ref[:, :, 0:w] = (jnp.where(hit0, p0, 0.0)
                           + jnp.where(hit1, p1, 0.0))
    zt = jnp.zeros((tm, n_e, cap - w), jnp.float32)
    disp_ref[:, :, w:cap] = zt
    comb_ref[:, :, w:cap] = zt

    @pl.when(i == 0)
    def _():
        mean_p = psum_ref[...] * (1.0 / n_tokens)
        aux = jnp.sum(mean_p * jnp.log(mean_p * n_e + 1e-9))
        aux_ref[...] = aux.reshape(1, 1)


def kernel(hidden_states, W1, b1, W2, b2):
    b, s, h = hidden_states.shape
    e = W2.shape[1]
    n = b * s
    cap = int(b * s * _CAP_FACTOR * _TOPK / e)
    x = hidden_states.reshape(n, h)

    tb = 256
    probs, probsP, pres16, psum = pl.pallas_call(
        _router_body,
        grid=(n // tb,),
        in_specs=[
            pl.BlockSpec((tb, h), lambda i: (i, 0)),
            pl.BlockSpec((h, h), lambda i: (0, 0)),
            pl.BlockSpec((1, h), lambda i: (0, 0)),
            pl.BlockSpec((h, e), lambda i: (0, 0)),
            pl.BlockSpec((1, e), lambda i: (0, 0)),
        ],
        out_specs=[
            pl.BlockSpec((tb, e), lambda i: (i, 0)),
            pl.BlockSpec((2, e, 128), lambda i: (i, 0, 0)),
            pl.BlockSpec((1, 16), lambda i: (0, 0)),
            pl.BlockSpec((1, e), lambda i: (0, 0)),
        ],
        out_shape=[
            jax.ShapeDtypeStruct((n, e), jnp.float32),
            jax.ShapeDtypeStruct((n // 128, e, 128), jnp.float32),
            jax.ShapeDtypeStruct((1, 16), jnp.float32),
            jax.ShapeDtypeStruct((1, e), jnp.float32),
        ],
    )(x, W1, b1.reshape(1, h), W2, b2.reshape(1, e))

    meta = _make_sc_router(n, e)(probsP, pres16.reshape(16))

    tm = 128
    disp, comb, aux = pl.pallas_call(
        functools.partial(_scatter_body, cap=cap, n_tokens=n),
        grid=(n // tm,),
        in_specs=[
            pl.BlockSpec((1, 8, tm), lambda i: (i, 0, 0)),
            pl.BlockSpec((1, e), lambda i: (0, 0)),
        ],
        out_specs=[
            pl.BlockSpec((tm, e, cap), lambda i: (i, 0, 0)),
            pl.BlockSpec((tm, e, cap), lambda i: (i, 0, 0)),
            pl.BlockSpec((1, 1), lambda i: (0, 0)),
        ],
        out_shape=[
            jax.ShapeDtypeStruct((n, e, cap), jnp.float32),
            jax.ShapeDtypeStruct((n, e, cap), jnp.float32),
            jax.ShapeDtypeStruct((1, 1), jnp.float32),
        ],
    )(meta, psum)

    return (disp.reshape(b, s, e, cap), comb.reshape(b, s, e, cap),
            probs.reshape(b, s, e), aux[0, 0])


# speedup vs baseline: 1.0895x; 1.0020x over previous
import functools

import jax
import jax.numpy as jnp
from jax import lax
from jax.experimental import pallas as pl
from jax.experimental.pallas import tpu as pltpu
from jax.experimental.pallas import tpu_sc as plsc

_TOPK = 2
_CAP_FACTOR = 1.5

_NC = 2
_NS = 16
_L = 16


def _router_body(x_ref, w1_ref, b1_ref, w2_ref, b2_ref,
                 probs_ref, probsP_ref, pres_ref, psum_ref):
    i = pl.program_id(0)
    x = x_ref[...]
    h = jnp.maximum(
        jnp.dot(x, w1_ref[...], preferred_element_type=jnp.float32)
        + b1_ref[...], 0.0)
    logits = (jnp.dot(h, w2_ref[...], preferred_element_type=jnp.float32)
              + b2_ref[...])
    m = jnp.max(logits, axis=-1, keepdims=True)
    ex = jnp.exp(logits - m)
    probs = ex / jnp.sum(ex, axis=-1, keepdims=True)
    probs_ref[...] = probs

    tn, n_e = probs.shape
    eio = jax.lax.broadcasted_iota(
        jnp.int32, (tn, n_e), 1).astype(jnp.float32)
    p0 = jnp.max(probs, axis=-1, keepdims=True)
    e0 = jnp.min(jnp.where(probs == p0, eio, float(n_e)),
                 axis=-1, keepdims=True)
    blk_pres = jnp.max((eio == e0).astype(jnp.float32),
                       axis=0, keepdims=True)
    blk_psum = jnp.sum(probs, axis=0, keepdims=True)
    pad = jnp.zeros((1, 16 - n_e), jnp.float32)
    blk_pres16 = jnp.concatenate([blk_pres, pad], axis=1)
    probsP_ref[...] = jnp.transpose(probs.reshape(2, 128, n_e), (0, 2, 1))

    @pl.when(i == 0)
    def _():
        pres_ref[...] = blk_pres16
        psum_ref[...] = blk_psum

    @pl.when(i > 0)
    def _():
        pres_ref[...] = jnp.maximum(pres_ref[...], blk_pres16)
        psum_ref[...] = psum_ref[...] + blk_psum


def _make_sc_router(n, n_e):
    tok_per_p = 128
    npair = n // tok_per_p
    groups = tok_per_p // _L
    mesh = plsc.VectorSubcoreMesh(core_axis_name="c", subcore_axis_name="s")

    def _route_chunk(probsP_hbm, pres_hbm, meta_hbm, pbuf, mchunk,
                     presv, wid):
        pltpu.sync_copy(probsP_hbm.at[wid], pbuf)
        pltpu.sync_copy(pres_hbm.at[0], presv)
        presvec = presv[...]
        for g in range(groups):
            pv = [pbuf[e, pl.ds(g * _L, _L)] for e in range(n_e)]
            p0 = pv[0]
            e0 = jnp.zeros((_L,), jnp.float32)
            for e in range(1, n_e):
                mgt = pv[e] > p0
                p0 = jnp.where(mgt, pv[e], p0)
                e0 = jnp.where(mgt, float(e), e0)
            p1 = jnp.full((_L,), -1.0, jnp.float32)
            e1 = jnp.zeros((_L,), jnp.float32)
            for e in range(n_e):
                mgt = (e0 != float(e)) & (pv[e] > p1)
                p1 = jnp.where(mgt, pv[e], p1)
                e1 = jnp.where(mgt, float(e), e1)
            s = p0 + p1
            pos1 = presvec.at[e1.astype(jnp.int32)].get(
                mode="promise_in_bounds")
            sl = pl.ds(g * _L, _L)
            mchunk[0, sl] = e0
            mchunk[1, sl] = e1
            mchunk[2, sl] = p0 / s
            mchunk[3, sl] = p1 / s
            mchunk[4, sl] = pos1
            zl = jnp.zeros((_L,), jnp.float32)
            mchunk[5, sl] = zl
            mchunk[6, sl] = zl
            mchunk[7, sl] = zl
        pltpu.sync_copy(mchunk, meta_hbm.at[wid])

    @functools.partial(
        pl.kernel, mesh=mesh,
        out_type=jax.ShapeDtypeStruct((npair, 8, tok_per_p), jnp.float32),
        scratch_types=[
            pltpu.VMEM((n_e, tok_per_p), jnp.float32),
            pltpu.VMEM((8, tok_per_p), jnp.float32),
            pltpu.VMEM((16,), jnp.float32),
        ],
    )
    def sc_route(probsP_hbm, pres_hbm, meta_hbm, pbuf, mchunk, presv):
        wid = lax.axis_index("s") * _NC + lax.axis_index("c")

        @pl.when(wid < npair)
        def _():
            _route_chunk(probsP_hbm, pres_hbm, meta_hbm, pbuf, mchunk,
                         presv, wid)

    return sc_route


def _scatter_body(meta_ref, psum_ref, disp_ref, comb_ref, aux_ref,
                  *, cap, n_tokens):
    i = pl.program_id(0)
    meta = jnp.transpose(meta_ref[0, :, :])
    tm = meta.shape[0]
    n_e = psum_ref.shape[1]
    e0 = meta[:, 0].reshape(tm, 1, 1)
    e1 = meta[:, 1].reshape(tm, 1, 1)
    p0 = meta[:, 2].reshape(tm, 1, 1)
    p1 = meta[:, 3].reshape(tm, 1, 1)
    pos1 = meta[:, 4].reshape(tm, 1, 1)

    w = 128
    e_ax = jax.lax.broadcasted_iota(
        jnp.int32, (tm, n_e, w), 1).astype(jnp.float32)
    c_ax = jax.lax.broadcasted_iota(
        jnp.int32, (tm, n_e, w), 2).astype(jnp.float32)
    hit0 = (e_ax == e0) & (c_ax == 0.0)
    hit1 = (e_ax == e1) & (c_ax == pos1)
    disp_ref[:, :, 0:w] = hit0.astype(jnp.float32) + hit1.astype(jnp.float32)
    comb_ref[:, :, 0:w] = (jnp.where(hit0, p0, 0.0)
                           + jnp.where(hit1, p1, 0.0))
    zt = jnp.zeros((tm, n_e, cap - w), jnp.float32)
    disp_ref[:, :, w:cap] = zt
    comb_ref[:, :, w:cap] = zt

    @pl.when(i == 0)
    def _():
        mean_p = psum_ref[...] * (1.0 / n_tokens)
        aux = jnp.sum(mean_p * jnp.log(mean_p * n_e + 1e-9))
        aux_ref[...] = aux.reshape(1, 1)


def kernel(hidden_states, W1, b1, W2, b2):
    b, s, h = hidden_states.shape
    e = W2.shape[1]
    n = b * s
    cap = int(b * s * _CAP_FACTOR * _TOPK / e)
    x = hidden_states.reshape(n, h)

    tb = 256
    probs, probsP, pres16, psum = pl.pallas_call(
        _router_body,
        grid=(n // tb,),
        in_specs=[
            pl.BlockSpec((tb, h), lambda i: (i, 0)),
            pl.BlockSpec((h, h), lambda i: (0, 0)),
            pl.BlockSpec((1, h), lambda i: (0, 0)),
            pl.BlockSpec((h, e), lambda i: (0, 0)),
            pl.BlockSpec((1, e), lambda i: (0, 0)),
        ],
        out_specs=[
            pl.BlockSpec((tb, e), lambda i: (i, 0)),
            pl.BlockSpec((2, e, 128), lambda i: (i, 0, 0)),
            pl.BlockSpec((1, 16), lambda i: (0, 0)),
            pl.BlockSpec((1, e), lambda i: (0, 0)),
        ],
        out_shape=[
            jax.ShapeDtypeStruct((n, e), jnp.float32),
            jax.ShapeDtypeStruct((n // 128, e, 128), jnp.float32),
            jax.ShapeDtypeStruct((1, 16), jnp.float32),
            jax.ShapeDtypeStruct((1, e), jnp.float32),
        ],
    )(x, W1, b1.reshape(1, h), W2, b2.reshape(1, e))

    meta = _make_sc_router(n, e)(probsP, pres16)

    tm = 128
    disp, comb, aux = pl.pallas_call(
        functools.partial(_scatter_body, cap=cap, n_tokens=n),
        grid=(n // tm,),
        in_specs=[
            pl.BlockSpec((1, 8, tm), lambda i: (i, 0, 0)),
            pl.BlockSpec((1, e), lambda i: (0, 0)),
        ],
        out_specs=[
            pl.BlockSpec((tm, e, cap), lambda i: (i, 0, 0)),
            pl.BlockSpec((tm, e, cap), lambda i: (i, 0, 0)),
            pl.BlockSpec((1, 1), lambda i: (0, 0)),
        ],
        out_shape=[
            jax.ShapeDtypeStruct((n, e, cap), jnp.float32),
            jax.ShapeDtypeStruct((n, e, cap), jnp.float32),
            jax.ShapeDtypeStruct((1, 1), jnp.float32),
        ],
    )(meta, psum)

    return (disp.reshape(b, s, e, cap), comb.reshape(b, s, e, cap),
            probs.reshape(b, s, e), aux[0, 0])
